# Initial kernel scaffold; baseline (speedup 1.0000x reference)
import jax
import jax.numpy as jnp
from jax.experimental import pallas as pl


def kernel(hidden_states, weight, bias, top_k):
    raise NotImplementedError("write your pallas kernel here")


# speedup vs baseline: 5.1457x; 5.1457x over previous
import functools

import jax
import jax.numpy as jnp
from jax.experimental import pallas as pl
from jax.experimental.pallas import tpu as pltpu

_T = 4 * 4096
_D = 4096
_E = 64
_K = 8
_BT = 512


def _router_kernel(scale_ref, h_ref, wt_ref, b_ref, o_ref):
    logits = jnp.dot(h_ref[...], wt_ref[...], preferred_element_type=jnp.float32)
    logits = logits + b_ref[...]

    lane = jax.lax.broadcasted_iota(jnp.int32, logits.shape, 1)
    x = logits
    mask = jnp.zeros(logits.shape, jnp.bool_)
    m0 = None
    for _ in range(_K):
        m = jnp.max(x, axis=1, keepdims=True)
        if m0 is None:
            m0 = m
        first = jnp.min(jnp.where(x == m, lane, _E), axis=1, keepdims=True)
        sel = lane == first
        mask = jnp.logical_or(mask, sel)
        x = jnp.where(sel, -jnp.inf, x)

    e = jnp.where(mask, jnp.exp(logits - m0), 0.0)
    s = jnp.sum(e, axis=1, keepdims=True)
    o_ref[...] = e * (scale_ref[0] / s)


def kernel(hidden_states, weight, bias, top_k):
    wt = weight.T
    bias2 = bias.reshape(1, _E)
    scale = jnp.asarray(top_k - (_K - 1), jnp.float32).reshape(1)

    grid = (_T // _BT,)
    out = pl.pallas_call(
        _router_kernel,
        grid=grid,
        in_specs=[
            pl.BlockSpec(memory_space=pltpu.SMEM),
            pl.BlockSpec((_BT, _D), lambda i: (i, 0)),
            pl.BlockSpec((_D, _E), lambda i: (0, 0)),
            pl.BlockSpec((1, _E), lambda i: (0, 0)),
        ],
        out_specs=pl.BlockSpec((_BT, _E), lambda i: (i, 0)),
        out_shape=jax.ShapeDtypeStruct((_T, _E), jnp.float32),
        compiler_params=pltpu.CompilerParams(
            dimension_semantics=("arbitrary",),
        ),
    )(scale, hidden_states, wt, bias2)
    return out


# speedup vs baseline: 5.7624x; 1.1199x over previous
import functools

import jax
import jax.numpy as jnp
from jax.experimental import pallas as pl
from jax.experimental.pallas import tpu as pltpu

_T = 4 * 4096
_D = 4096
_E = 64
_K = 8
_BT = 512


def _router_kernel(scale_ref, h_ref, wt_ref, b_ref, o_ref):
    logits = jnp.dot(h_ref[...], wt_ref[...], preferred_element_type=jnp.float32)
    logits = logits + b_ref[...]

    lane = jax.lax.broadcasted_iota(jnp.int32, logits.shape, 1).astype(jnp.float32)
    x = logits
    mask = jnp.zeros(logits.shape, jnp.bool_)
    m0 = None
    for _ in range(_K):
        m = jnp.max(x, axis=1, keepdims=True)
        if m0 is None:
            m0 = m
        first = jnp.min(jnp.where(x == m, lane, float(_E)), axis=1, keepdims=True)
        sel = lane == first
        mask = jnp.logical_or(mask, sel)
        x = jnp.where(sel, -jnp.inf, x)

    e = jnp.where(mask, jnp.exp(logits - m0), 0.0)
    s = jnp.sum(e, axis=1, keepdims=True)
    o_ref[...] = e * (scale_ref[0] / s)


def kernel(hidden_states, weight, bias, top_k):
    wt = weight.T
    bias2 = bias.reshape(1, _E)
    scale = jnp.asarray(top_k - (_K - 1), jnp.float32).reshape(1)

    grid = (_T // _BT,)
    out = pl.pallas_call(
        _router_kernel,
        grid=grid,
        in_specs=[
            pl.BlockSpec(memory_space=pltpu.SMEM),
            pl.BlockSpec((_BT, _D), lambda i: (i, 0)),
            pl.BlockSpec((_D, _E), lambda i: (0, 0)),
            pl.BlockSpec((1, _E), lambda i: (0, 0)),
        ],
        out_specs=pl.BlockSpec((_BT, _E), lambda i: (i, 0)),
        out_shape=jax.ShapeDtypeStruct((_T, _E), jnp.float32),
        compiler_params=pltpu.CompilerParams(
            dimension_semantics=("parallel",),
        ),
    )(scale, hidden_states, wt, bias2)
    return out


# speedup vs baseline: 6.4449x; 1.1184x over previous
import functools

import jax
import jax.numpy as jnp
from jax.experimental import pallas as pl
from jax.experimental.pallas import tpu as pltpu

_T = 4 * 4096
_D = 4096
_E = 64
_K = 8
_BT = 1024


def _router_kernel(scale_ref, h_ref, wt_ref, b_ref, o_ref):
    logits = jnp.dot(h_ref[...], wt_ref[...], preferred_element_type=jnp.float32)
    logits = logits + b_ref[...]

    lane = jax.lax.broadcasted_iota(jnp.int32, logits.shape, 1).astype(jnp.float32)
    x = logits
    mask = jnp.zeros(logits.shape, jnp.bool_)
    m0 = None
    for _ in range(_K):
        m = jnp.max(x, axis=1, keepdims=True)
        if m0 is None:
            m0 = m
        first = jnp.min(jnp.where(x == m, lane, float(_E)), axis=1, keepdims=True)
        sel = lane == first
        mask = jnp.logical_or(mask, sel)
        x = jnp.where(sel, -jnp.inf, x)

    e = jnp.where(mask, jnp.exp(logits - m0), 0.0)
    s = jnp.sum(e, axis=1, keepdims=True)
    o_ref[...] = e * (scale_ref[0] / s)


def kernel(hidden_states, weight, bias, top_k):
    wt = weight.T
    bias2 = bias.reshape(1, _E)
    scale = jnp.asarray(top_k - (_K - 1), jnp.float32).reshape(1)

    grid = (_T // _BT,)
    out = pl.pallas_call(
        _router_kernel,
        grid=grid,
        in_specs=[
            pl.BlockSpec(memory_space=pltpu.SMEM),
            pl.BlockSpec((_BT, _D), lambda i: (i, 0)),
            pl.BlockSpec((_D, _E), lambda i: (0, 0)),
            pl.BlockSpec((1, _E), lambda i: (0, 0)),
        ],
        out_specs=pl.BlockSpec((_BT, _E), lambda i: (i, 0)),
        out_shape=jax.ShapeDtypeStruct((_T, _E), jnp.float32),
        compiler_params=pltpu.CompilerParams(
            dimension_semantics=("parallel",),
        ),
    )(scale, hidden_states, wt, bias2)
    return out


# speedup vs baseline: 6.4529x; 1.0012x over previous
import functools

import jax
import jax.numpy as jnp
from jax.experimental import pallas as pl
from jax.experimental.pallas import tpu as pltpu

_T = 4 * 4096
_D = 4096
_E = 64
_K = 8
_BT = 1024


def _router_kernel(scale_ref, h_ref, wt_ref, b_ref, o_ref):
    logits = jnp.dot(h_ref[...], wt_ref[...], preferred_element_type=jnp.float32)
    logits = logits + b_ref[...]

    lane = jax.lax.broadcasted_iota(jnp.int32, logits.shape, 1).astype(jnp.float32)
    x = logits
    m0 = None
    for _ in range(_K):
        m = jnp.max(x, axis=1, keepdims=True)
        if m0 is None:
            m0 = m
        first = jnp.min(jnp.where(x == m, lane, float(_E)), axis=1, keepdims=True)
        x = jnp.where(lane == first, -jnp.inf, x)

    e = jnp.where(x == -jnp.inf, jnp.exp(logits - m0), 0.0)
    s = jnp.sum(e, axis=1, keepdims=True)
    o_ref[...] = e * (scale_ref[0] / s)


def kernel(hidden_states, weight, bias, top_k):
    wt = weight.T
    bias2 = bias.reshape(1, _E)
    scale = jnp.asarray(top_k - (_K - 1), jnp.float32).reshape(1)

    grid = (_T // _BT,)
    out = pl.pallas_call(
        _router_kernel,
        grid=grid,
        in_specs=[
            pl.BlockSpec(memory_space=pltpu.SMEM),
            pl.BlockSpec((_BT, _D), lambda i: (i, 0)),
            pl.BlockSpec((_D, _E), lambda i: (0, 0)),
            pl.BlockSpec((1, _E), lambda i: (0, 0)),
        ],
        out_specs=pl.BlockSpec((_BT, _E), lambda i: (i, 0)),
        out_shape=jax.ShapeDtypeStruct((_T, _E), jnp.float32),
        compiler_params=pltpu.CompilerParams(
            dimension_semantics=("parallel",),
        ),
    )(scale, hidden_states, wt, bias2)
    return out
